# baseline (speedup 1.0000x reference)
import dataclasses
import functools

import jax
import jax.numpy as jnp
from jax import lax
from jax.experimental import pallas as pl
from jax.experimental.pallas import tpu as pltpu
from jax.experimental.pallas import tpu_sc as plsc

_NUM_NODES = 10000
_D = 128
_E = 320000
_NC = 2
_NS = 16
_NW = _NC * _NS
_EPW = _E // _NW
_W = 200
_NWIN = _EPW // _W
_L = 16

_PERM_DNUMS = lax.GatherDimensionNumbers(
    offset_dims=(), collapsed_slice_dims=(0,), start_index_map=(0,))


def _permute(x, idx):
    return lax.gather(x, idx[:, None], _PERM_DNUMS, slice_sizes=(1,),
                      mode=lax.GatherScatterMode.PROMISE_IN_BOUNDS)


def _edge_scores(z, cidx_all, src, w):
    mesh = plsc.VectorSubcoreMesh(core_axis_name="c", subcore_axis_name="s")
    cp = pltpu.CompilerParams()
    if "needs_layout_passes" in pltpu.CompilerParams.__dataclass_fields__:
        cp = dataclasses.replace(cp, needs_layout_passes=False)
    if "use_tc_tiling_on_sc" in pltpu.CompilerParams.__dataclass_fields__:
        cp = dataclasses.replace(cp, use_tc_tiling_on_sc=False)

    @functools.partial(
        pl.kernel,
        compiler_params=cp,
        out_type=jax.ShapeDtypeStruct((_E,), jnp.float32),
        mesh=mesh,
        scratch_types=[
            pltpu.VMEM((_NUM_NODES,), jnp.float32),
            pltpu.VMEM((2 * _EPW,), jnp.int32),
            pltpu.VMEM((_EPW,), jnp.int32),
            pltpu.VMEM((_EPW,), jnp.float32),
            pltpu.VMEM((2 * _W, _D // 2), jnp.int32),
            pltpu.VMEM((2 * _W, _D // 2), jnp.int32),
            pltpu.SemaphoreType.DMA,
            pltpu.SemaphoreType.DMA,
        ],
    )
    def k(z_hbm, cidx_hbm, src_hbm, w_hbm, out_hbm,
          w_v, cidx, sidx, outv, rows_a, rows_b, sem_a, sem_b):
        wid = lax.axis_index("s") * _NC + lax.axis_index("c")
        base = wid * _EPW
        pltpu.sync_copy(w_hbm, w_v)
        pltpu.sync_copy(cidx_hbm.at[pl.ds(2 * base, 2 * _EPW)], cidx)
        pltpu.sync_copy(src_hbm.at[pl.ds(base, _EPW)], sidx)

        def copies(win, rows, sem):
            return pltpu.make_async_copy(
                z_hbm.at[cidx.at[pl.ds(2 * _W * win, 2 * _W)]], rows, sem)

        def issue(win, rows, sem):
            copies(win, rows, sem).start()

        lane = lax.iota(jnp.int32, _L)
        perms = [lane ^ sh for sh in (1, 2, 4, 8)]
        mask0 = lane == 0

        _P = 8

        def compute(win, rows, sem):
            copies(win, rows, sem).wait()
            woff = win * _W

            def dots(e):
                accs = []
                for j in range(_P):
                    acc = None
                    for kk in range(_D // (2 * _L)):
                        si = rows[e + j, pl.ds(kk * _L, _L)]
                        di = rows[_W + e + j, pl.ds(kk * _L, _L)]
                        pr = (plsc.bitcast(si, jnp.bfloat16)
                              * plsc.bitcast(di, jnp.bfloat16))
                        lo, hi = plsc.unpack(
                            pr, format=plsc.PackFormat.INTERLEAVED)
                        acc = lo + hi if acc is None else acc + lo + hi
                    accs.append(acc)
                return tuple(accs)

            def reduce_store(e, accs, mask):
                for j in range(_P):
                    acc = accs[j]
                    for p in perms:
                        acc = acc + _permute(acc, p)
                    idx = jnp.broadcast_to(e + j, (_L,))
                    plsc.store_scatter(outv, [idx], acc, mask=mask)

            @pl.loop(0, _W, step=_P)
            def _e4(e):
                reduce_store(woff + e, dots(e), mask0)

        issue(0, rows_a, sem_a)

        @pl.loop(0, _NWIN - 2, step=2)
        def _win(wn):
            issue(wn + 1, rows_b, sem_b)
            compute(wn, rows_a, sem_a)
            issue(wn + 2, rows_a, sem_a)
            compute(wn + 1, rows_b, sem_b)

        issue(_NWIN - 1, rows_b, sem_b)
        compute(_NWIN - 2, rows_a, sem_a)
        compute(_NWIN - 1, rows_b, sem_b)

        @pl.loop(0, _EPW, step=_L)
        def _wgt(e):
            sl = pl.ds(e, _L)
            x = outv[sl] * plsc.load_gather(w_v, [sidx[sl]])
            outv[sl] = 1.0 / (1.0 + jnp.exp(-x))

        pltpu.sync_copy(outv, out_hbm.at[pl.ds(base, _EPW)])

    return k(z, cidx_all, src, w)


def kernel(z, edge_index, weight):
    ei = edge_index.astype(jnp.int32)
    cidx = (ei.reshape(2, _NW, _NWIN, _W)
            .transpose(1, 2, 0, 3)
            .reshape(-1))
    zi = lax.bitcast_convert_type(
        z.astype(jnp.bfloat16).reshape(_NUM_NODES, _D // 2, 2), jnp.int32)
    return _edge_scores(zi, cidx, ei[0], weight)


# speedup vs baseline: 1.1211x; 1.1211x over previous
import dataclasses
import functools

import jax
import jax.numpy as jnp
from jax import lax
from jax.experimental import pallas as pl
from jax.experimental.pallas import tpu as pltpu
from jax.experimental.pallas import tpu_sc as plsc

_NUM_NODES = 10000
_D = 128
_E = 320000
_NC = 2
_NS = 16
_NW = _NC * _NS
_EPW = _E // _NW
_W = 200
_NWIN = _EPW // _W
_L = 16

_PERM_DNUMS = lax.GatherDimensionNumbers(
    offset_dims=(), collapsed_slice_dims=(0,), start_index_map=(0,))


def _permute(x, idx):
    return lax.gather(x, idx[:, None], _PERM_DNUMS, slice_sizes=(1,),
                      mode=lax.GatherScatterMode.PROMISE_IN_BOUNDS)


def _edge_scores(z, src, dst, w):
    mesh = plsc.VectorSubcoreMesh(core_axis_name="c", subcore_axis_name="s")
    cp = pltpu.CompilerParams()
    if "needs_layout_passes" in pltpu.CompilerParams.__dataclass_fields__:
        cp = dataclasses.replace(cp, needs_layout_passes=False)
    if "use_tc_tiling_on_sc" in pltpu.CompilerParams.__dataclass_fields__:
        cp = dataclasses.replace(cp, use_tc_tiling_on_sc=False)

    @functools.partial(
        pl.kernel,
        compiler_params=cp,
        out_type=jax.ShapeDtypeStruct((_E,), jnp.float32),
        mesh=mesh,
        scratch_types=[
            pltpu.VMEM((_NUM_NODES,), jnp.float32),
            pltpu.VMEM((_EPW,), jnp.int32),
            pltpu.VMEM((_EPW,), jnp.int32),
            pltpu.VMEM((_EPW,), jnp.float32),
            pltpu.VMEM((_W, _D // 2), jnp.int32),
            pltpu.VMEM((_W, _D // 2), jnp.int32),
            pltpu.VMEM((_W, _D // 2), jnp.int32),
            pltpu.VMEM((_W, _D // 2), jnp.int32),
            pltpu.SemaphoreType.DMA,
            pltpu.SemaphoreType.DMA,
            pltpu.SemaphoreType.DMA,
            pltpu.SemaphoreType.DMA,
        ],
    )
    def k(z_hbm, src_hbm, dst_hbm, w_hbm, out_hbm,
          w_v, sidx, didx, outv, srows_a, drows_a, srows_b, drows_b,
          sem_sa, sem_da, sem_sb, sem_db):
        wid = lax.axis_index("s") * _NC + lax.axis_index("c")
        base = wid * _EPW
        pltpu.sync_copy(w_hbm, w_v)
        pltpu.sync_copy(src_hbm.at[pl.ds(base, _EPW)], sidx)
        pltpu.sync_copy(dst_hbm.at[pl.ds(base, _EPW)], didx)

        _SPLITS = ((0, 104), (104, 96))

        def copies(win, srows, drows, sem_s, sem_d):
            off = win * _W
            out = []
            for rows, idx, sem in ((srows, sidx, sem_s),
                                   (drows, didx, sem_d)):
                for h0, hl in _SPLITS:
                    out.append(pltpu.make_async_copy(
                        z_hbm.at[idx.at[pl.ds(off + h0, hl)]],
                        rows.at[pl.ds(h0, hl)], sem))
            return out

        def issue(win, srows, drows, sem_s, sem_d):
            for c in copies(win, srows, drows, sem_s, sem_d):
                c.start()

        lane = lax.iota(jnp.int32, _L)
        perms = [lane ^ sh for sh in (1, 2, 4, 8)]
        mask0 = lane == 0

        _P = 8

        def compute(win, srows, drows, sem_s, sem_d):
            for c in copies(win, srows, drows, sem_s, sem_d):
                c.wait()
            woff = win * _W

            def dots(e):
                accs = []
                for j in range(_P):
                    acc = None
                    for kk in range(_D // (2 * _L)):
                        si = srows[e + j, pl.ds(kk * _L, _L)]
                        di = drows[e + j, pl.ds(kk * _L, _L)]
                        pr = (plsc.bitcast(si, jnp.bfloat16)
                              * plsc.bitcast(di, jnp.bfloat16))
                        lo, hi = plsc.unpack(
                            pr, format=plsc.PackFormat.INTERLEAVED)
                        acc = lo + hi if acc is None else acc + lo + hi
                    accs.append(acc)
                return tuple(accs)

            def reduce_store(e, accs, mask):
                for j in range(_P):
                    acc = accs[j]
                    for p in perms:
                        acc = acc + _permute(acc, p)
                    idx = jnp.broadcast_to(e + j, (_L,))
                    plsc.store_scatter(outv, [idx], acc, mask=mask)

            @pl.loop(0, _W, step=_P)
            def _e4(e):
                reduce_store(woff + e, dots(e), mask0)

        issue(0, srows_a, drows_a, sem_sa, sem_da)

        @pl.loop(0, _NWIN - 2, step=2)
        def _win(wn):
            issue(wn + 1, srows_b, drows_b, sem_sb, sem_db)
            compute(wn, srows_a, drows_a, sem_sa, sem_da)
            issue(wn + 2, srows_a, drows_a, sem_sa, sem_da)
            compute(wn + 1, srows_b, drows_b, sem_sb, sem_db)

        issue(_NWIN - 1, srows_b, drows_b, sem_sb, sem_db)
        compute(_NWIN - 2, srows_a, drows_a, sem_sa, sem_da)
        compute(_NWIN - 1, srows_b, drows_b, sem_sb, sem_db)

        @pl.loop(0, _EPW, step=_L)
        def _wgt(e):
            sl = pl.ds(e, _L)
            x = outv[sl] * plsc.load_gather(w_v, [sidx[sl]])
            outv[sl] = 1.0 / (1.0 + jnp.exp(-x))

        pltpu.sync_copy(outv, out_hbm.at[pl.ds(base, _EPW)])

    return k(z, src, dst, w)


def kernel(z, edge_index, weight):
    ei = edge_index.astype(jnp.int32)
    zi = lax.bitcast_convert_type(
        z.astype(jnp.bfloat16).reshape(_NUM_NODES, _D // 2, 2), jnp.int32)
    return _edge_scores(zi, ei[0], ei[1], weight)


# speedup vs baseline: 1.1354x; 1.0128x over previous
import dataclasses
import functools

import jax
import jax.numpy as jnp
from jax import lax
from jax.experimental import pallas as pl
from jax.experimental.pallas import tpu as pltpu
from jax.experimental.pallas import tpu_sc as plsc

_NUM_NODES = 10000
_D = 128
_E = 320000
_NC = 2
_NS = 16
_NW = _NC * _NS
_EPW = _E // _NW
_W = 200
_NWIN = _EPW // _W
_L = 16

_PERM_DNUMS = lax.GatherDimensionNumbers(
    offset_dims=(), collapsed_slice_dims=(0,), start_index_map=(0,))


def _permute(x, idx):
    return lax.gather(x, idx[:, None], _PERM_DNUMS, slice_sizes=(1,),
                      mode=lax.GatherScatterMode.PROMISE_IN_BOUNDS)


def _edge_scores(z, src, dst, w):
    mesh = plsc.VectorSubcoreMesh(core_axis_name="c", subcore_axis_name="s")
    cp = pltpu.CompilerParams()
    if "needs_layout_passes" in pltpu.CompilerParams.__dataclass_fields__:
        cp = dataclasses.replace(cp, needs_layout_passes=False)
    if "use_tc_tiling_on_sc" in pltpu.CompilerParams.__dataclass_fields__:
        cp = dataclasses.replace(cp, use_tc_tiling_on_sc=False)

    @functools.partial(
        pl.kernel,
        compiler_params=cp,
        out_type=jax.ShapeDtypeStruct((_E,), jnp.float32),
        mesh=mesh,
        scratch_types=[
            pltpu.VMEM((_NUM_NODES,), jnp.float32),
            pltpu.VMEM((_EPW,), jnp.int32),
            pltpu.VMEM((_EPW,), jnp.int32),
            pltpu.VMEM((_EPW,), jnp.float32),
            pltpu.VMEM((_W, _D // 2), jnp.int32),
            pltpu.VMEM((_W, _D // 2), jnp.int32),
            pltpu.VMEM((_W, _D // 2), jnp.int32),
            pltpu.VMEM((_W, _D // 2), jnp.int32),
            pltpu.VMEM((_W, _D // 2), jnp.int32),
            pltpu.VMEM((_W, _D // 2), jnp.int32),
            pltpu.SemaphoreType.DMA,
            pltpu.SemaphoreType.DMA,
            pltpu.SemaphoreType.DMA,
            pltpu.SemaphoreType.DMA,
            pltpu.SemaphoreType.DMA,
            pltpu.SemaphoreType.DMA,
        ],
    )
    def k(z_hbm, src_hbm, dst_hbm, w_hbm, out_hbm,
          w_v, sidx, didx, outv, srows_a, drows_a, srows_b, drows_b,
          srows_c, drows_c, sem_sa, sem_da, sem_sb, sem_db,
          sem_sc, sem_dc):
        wid = lax.axis_index("s") * _NC + lax.axis_index("c")
        base = wid * _EPW
        pltpu.sync_copy(w_hbm, w_v)
        pltpu.sync_copy(src_hbm.at[pl.ds(base, _EPW)], sidx)
        pltpu.sync_copy(dst_hbm.at[pl.ds(base, _EPW)], didx)

        def copies(win, srows, drows, sem_s, sem_d):
            off = win * _W
            cs = pltpu.make_async_copy(
                z_hbm.at[sidx.at[pl.ds(off, _W)]], srows, sem_s)
            cd = pltpu.make_async_copy(
                z_hbm.at[didx.at[pl.ds(off, _W)]], drows, sem_d)
            return cs, cd

        def issue(win, srows, drows, sem_s, sem_d):
            cs, cd = copies(win, srows, drows, sem_s, sem_d)
            cs.start()
            cd.start()

        lane = lax.iota(jnp.int32, _L)
        perms = [lane ^ sh for sh in (1, 2, 4, 8)]
        mask0 = lane == 0

        _P = 8

        def compute(win, srows, drows, sem_s, sem_d):
            cs, cd = copies(win, srows, drows, sem_s, sem_d)
            cs.wait()
            cd.wait()
            woff = win * _W

            def dots(e):
                accs = []
                for j in range(_P):
                    acc = None
                    for kk in range(_D // (2 * _L)):
                        si = srows[e + j, pl.ds(kk * _L, _L)]
                        di = drows[e + j, pl.ds(kk * _L, _L)]
                        pr = (plsc.bitcast(si, jnp.bfloat16)
                              * plsc.bitcast(di, jnp.bfloat16))
                        lo, hi = plsc.unpack(
                            pr, format=plsc.PackFormat.INTERLEAVED)
                        acc = lo + hi if acc is None else acc + lo + hi
                    accs.append(acc)
                return tuple(accs)

            def reduce_store(e, accs, mask):
                for j in range(_P):
                    acc = accs[j]
                    for p in perms:
                        acc = acc + _permute(acc, p)
                    idx = jnp.broadcast_to(e + j, (_L,))
                    plsc.store_scatter(outv, [idx], acc, mask=mask)

            @pl.loop(0, _W, step=_P)
            def _e4(e):
                reduce_store(woff + e, dots(e), mask0)

        bufs = ((srows_a, drows_a, sem_sa, sem_da),
                (srows_b, drows_b, sem_sb, sem_db),
                (srows_c, drows_c, sem_sc, sem_dc))

        issue(0, *bufs[0])
        issue(1, *bufs[1])

        @pl.loop(0, _NWIN - 2, step=3)
        def _win(wn):
            issue(wn + 2, *bufs[2])
            compute(wn, *bufs[0])
            issue(wn + 3, *bufs[0])
            compute(wn + 1, *bufs[1])
            issue(wn + 4, *bufs[1])
            compute(wn + 2, *bufs[2])

        compute(_NWIN - 2, *bufs[0])
        compute(_NWIN - 1, *bufs[1])

        @pl.loop(0, _EPW, step=_L)
        def _wgt(e):
            sl = pl.ds(e, _L)
            x = outv[sl] * plsc.load_gather(w_v, [sidx[sl]])
            outv[sl] = 1.0 / (1.0 + jnp.exp(-x))

        pltpu.sync_copy(outv, out_hbm.at[pl.ds(base, _EPW)])

    return k(z, src, dst, w)


def kernel(z, edge_index, weight):
    ei = edge_index.astype(jnp.int32)
    zi = lax.bitcast_convert_type(
        z.astype(jnp.bfloat16).reshape(_NUM_NODES, _D // 2, 2), jnp.int32)
    return _edge_scores(zi, ei[0], ei[1], weight)
